# Initial kernel scaffold; baseline (speedup 1.0000x reference)
import jax
import jax.numpy as jnp
from jax import lax
from jax.experimental import pallas as pl
from jax.experimental.pallas import tpu as pltpu
from jax.experimental.pallas import tpu_sc as plsc

_N = 10000
_E = 320000
_FIN = 128
_H = 256
_B = 64

_NC, _NS, _L = 2, 16, 16
_NPAD = 10240
_RPT = _NPAD // _NS
_CH = 128
_EPAD = 323584
_EPW = _EPAD // (_NC * _NS)
_EPS = _EPAD // _NS
_RB = 512


def _sc_mesh():
    return plsc.VectorSubcoreMesh(
        core_axis_name="c", subcore_axis_name="s",
        num_cores=_NC, num_subcores=_NS)


def _deg_body(dst_hbm, out_hbm, idxd, ones_v, zbuf, acc):
    c = lax.axis_index("c")
    s = lax.axis_index("s")
    w = s * _NC + c
    zeros = jnp.zeros((_L,), jnp.float32)
    ones = jnp.ones((_L,), jnp.float32)

    def _fill(i, carry):
        zbuf[i, pl.ds(0, _L)] = zeros
        return carry
    lax.fori_loop(0, _RPT, _fill, 0)

    def _fill1(i, carry):
        ones_v[i, pl.ds(0, _L)] = ones
        return carry
    lax.fori_loop(0, _CH, _fill1, 0)

    pltpu.sync_copy(zbuf, acc.at[pl.ds(s * _RPT, _RPT)])
    plsc.subcore_barrier()

    def _chunk(k, carry):
        base = w * _EPW + k * _CH
        pltpu.sync_copy(dst_hbm.at[pl.ds(base, _CH)], idxd)
        pltpu.sync_copy(ones_v, acc.at[idxd], add=True)
        return carry
    lax.fori_loop(0, _EPW // _CH, _chunk, 0)

    plsc.subcore_barrier()
    pltpu.sync_copy(acc.at[pl.ds(s * _RPT, _RPT)],
                    out_hbm.at[c, pl.ds(s * _RPT, _RPT)])


def _make_deg():
    return pl.kernel(
        _deg_body,
        out_type=jax.ShapeDtypeStruct((_NC, _NPAD, _L), jnp.float32),
        mesh=_sc_mesh(),
        scratch_types=[
            pltpu.VMEM((_CH,), jnp.int32),
            pltpu.VMEM((_CH, _L), jnp.float32),
            pltpu.VMEM((_RPT, _L), jnp.float32),
            pltpu.VMEM_SHARED((_NPAD, _L), jnp.float32),
        ],
    )


def _make_agg(Fh):
    def _core_work(s, xt_hbm, s_hbm, dinv_hbm, src_hbm, dst_hbm,
                   idxs, idxd, rows, xv, dv, acc, sem):
        pltpu.sync_copy(xt_hbm.at[pl.ds(s * _RPT, _RPT)], xv)
        pltpu.sync_copy(xv, acc.at[pl.ds(s * _RPT, _RPT)])
        plsc.subcore_barrier()

        def _chunk(k, carry):
            base = s * _EPS + k * _CH
            pltpu.sync_copy(src_hbm.at[pl.ds(base, _CH)], idxs)
            pltpu.async_copy(xt_hbm.at[idxs], rows, sem).wait()
            pltpu.sync_copy(dst_hbm.at[pl.ds(base, _CH)], idxd)
            pltpu.sync_copy(rows, acc.at[idxd], add=True)
            return carry
        lax.fori_loop(0, _EPS // _CH, _chunk, 0)

        plsc.subcore_barrier()
        pltpu.sync_copy(acc.at[pl.ds(s * _RPT, _RPT)], xv)
        pltpu.sync_copy(dinv_hbm.at[pl.ds(s * _RPT, _RPT)], dv)

        def _row(r, carry):
            db = plsc.load_gather(dv, [jnp.full((_L,), r, jnp.int32)])
            for j in range(Fh // _L):
                sl = pl.ds(j * _L, _L)
                xv[r, sl] = xv[r, sl] * db
            return carry
        lax.fori_loop(0, _RPT, _row, 0)
        pltpu.sync_copy(xv, s_hbm.at[pl.ds(s * _RPT, _RPT)])

    def _body(xta, xtb, dinv_hbm, src_hbm, dst_hbm, sa, sb,
              idxs, idxd, rows, xv, dv, acc, sem):
        c = lax.axis_index("c")
        s = lax.axis_index("s")

        @pl.when(c == 0)
        def _():
            _core_work(s, xta, sa, dinv_hbm, src_hbm, dst_hbm,
                       idxs, idxd, rows, xv, dv, acc, sem)

        @pl.when(c == 1)
        def _():
            _core_work(s, xtb, sb, dinv_hbm, src_hbm, dst_hbm,
                       idxs, idxd, rows, xv, dv, acc, sem)

    return pl.kernel(
        _body,
        out_type=(jax.ShapeDtypeStruct((_NPAD, Fh), jnp.float32),
                  jax.ShapeDtypeStruct((_NPAD, Fh), jnp.float32)),
        mesh=_sc_mesh(),
        scratch_types=[
            pltpu.VMEM((_CH,), jnp.int32),
            pltpu.VMEM((_CH,), jnp.int32),
            pltpu.VMEM((_CH, Fh), jnp.float32),
            pltpu.VMEM((_RPT, Fh), jnp.float32),
            pltpu.VMEM((_RPT,), jnp.float32),
            pltpu.VMEM_SHARED((_NPAD, Fh), jnp.float32),
            pltpu.SemaphoreType.DMA,
        ],
    )


def _stage_a_body(d0_ref, d1_ref, x_ref, dinv_ref, xa_ref, xb_ref):
    deg = d0_ref[...] + d1_ref[...] + 1.0
    dinv = lax.rsqrt(deg)
    dinv_ref[...] = dinv
    xt = x_ref[...] * dinv
    xa_ref[...] = xt[:, : _FIN // 2]
    xb_ref[...] = xt[:, _FIN // 2:]


_stage_a = pl.pallas_call(
    _stage_a_body,
    grid=(_NPAD // _RB,),
    in_specs=[
        pl.BlockSpec((_RB, 1), lambda i: (i, 0)),
        pl.BlockSpec((_RB, 1), lambda i: (i, 0)),
        pl.BlockSpec((_RB, _FIN), lambda i: (i, 0)),
    ],
    out_specs=[
        pl.BlockSpec((_RB, 1), lambda i: (i, 0)),
        pl.BlockSpec((_RB, _FIN // 2), lambda i: (i, 0)),
        pl.BlockSpec((_RB, _FIN // 2), lambda i: (i, 0)),
    ],
    out_shape=[
        jax.ShapeDtypeStruct((_NPAD, 1), jnp.float32),
        jax.ShapeDtypeStruct((_NPAD, _FIN // 2), jnp.float32),
        jax.ShapeDtypeStruct((_NPAD, _FIN // 2), jnp.float32),
    ],
)


def _stage_b_body(sa_ref, sb_ref, dinv_ref, w1_ref, b1_ref, xa_ref, xb_ref):
    agg = jnp.concatenate([sa_ref[...], sb_ref[...]], axis=1)
    h = jnp.dot(agg, w1_ref[...], preferred_element_type=jnp.float32)
    h = jnp.maximum(h + b1_ref[...], 0.0)
    xt2 = h * dinv_ref[...]
    xa_ref[...] = xt2[:, : _H // 2]
    xb_ref[...] = xt2[:, _H // 2:]


_stage_b = pl.pallas_call(
    _stage_b_body,
    grid=(_NPAD // _RB,),
    in_specs=[
        pl.BlockSpec((_RB, _FIN // 2), lambda i: (i, 0)),
        pl.BlockSpec((_RB, _FIN // 2), lambda i: (i, 0)),
        pl.BlockSpec((_RB, 1), lambda i: (i, 0)),
        pl.BlockSpec((_FIN, _H), lambda i: (0, 0)),
        pl.BlockSpec((1, _H), lambda i: (0, 0)),
    ],
    out_specs=[
        pl.BlockSpec((_RB, _H // 2), lambda i: (i, 0)),
        pl.BlockSpec((_RB, _H // 2), lambda i: (i, 0)),
    ],
    out_shape=[
        jax.ShapeDtypeStruct((_NPAD, _H // 2), jnp.float32),
        jax.ShapeDtypeStruct((_NPAD, _H // 2), jnp.float32),
    ],
)


def _stage_c_body(sa_ref, sb_ref, w2_ref, b2_ref, wc_ref, bc_ref, batch_ref,
                  out_ref, cnt_ref):
    i = pl.program_id(0)
    agg = jnp.concatenate([sa_ref[...], sb_ref[...]], axis=1)
    h2 = jnp.dot(agg, w2_ref[...], preferred_element_type=jnp.float32)
    h2 = jnp.maximum(h2 + b2_ref[...], 0.0)
    y = jnp.dot(h2, wc_ref[...], preferred_element_type=jnp.float32)
    b = batch_ref[0]
    rows = lax.broadcasted_iota(jnp.int32, (_B, _RB), 0)
    oh = (rows == b).astype(jnp.float32)
    part = jnp.dot(oh, y, preferred_element_type=jnp.float32)
    cpart = jnp.sum(oh, axis=1, keepdims=True)

    @pl.when(i == 0)
    def _():
        out_ref[...] = jnp.zeros_like(out_ref)
        cnt_ref[...] = jnp.zeros_like(cnt_ref)

    out_ref[...] += part
    cnt_ref[...] += cpart

    @pl.when(i == pl.num_programs(0) - 1)
    def _():
        out_ref[...] = (out_ref[...] / jnp.maximum(cnt_ref[...], 1.0)
                        + bc_ref[0, 0])


_stage_c = pl.pallas_call(
    _stage_c_body,
    grid=(_NPAD // _RB,),
    in_specs=[
        pl.BlockSpec((_RB, _H // 2), lambda i: (i, 0)),
        pl.BlockSpec((_RB, _H // 2), lambda i: (i, 0)),
        pl.BlockSpec((_H, _H), lambda i: (0, 0)),
        pl.BlockSpec((1, _H), lambda i: (0, 0)),
        pl.BlockSpec((_H, 8), lambda i: (0, 0)),
        pl.BlockSpec((1, 1), lambda i: (0, 0)),
        pl.BlockSpec((1, 1, _RB), lambda i: (i, 0, 0)),
    ],
    out_specs=pl.BlockSpec((_B, 8), lambda i: (0, 0)),
    out_shape=jax.ShapeDtypeStruct((_B, 8), jnp.float32),
    scratch_shapes=[pltpu.VMEM((_B, 1), jnp.float32)],
)


_sc_kernels = None


def _get_sc_kernels():
    global _sc_kernels
    if _sc_kernels is None:
        _sc_kernels = (_make_deg(), _make_agg(_FIN // 2), _make_agg(_H // 2))
    return _sc_kernels


def kernel(x, edge_index, batch, W1, b1, W2, b2, Wc, bc):
    deg_fn, agg64, agg128 = _get_sc_kernels()
    src = edge_index[0].astype(jnp.int32)
    dst = edge_index[1].astype(jnp.int32)
    pad_e = _EPAD - _E
    src_p = jnp.concatenate([src, jnp.zeros((pad_e,), jnp.int32)])
    dst_p = jnp.concatenate([dst, jnp.full((pad_e,), _N, jnp.int32)])
    x_p = jnp.pad(x, ((0, _NPAD - _N), (0, 0)))
    batch_p = jnp.concatenate(
        [batch.astype(jnp.int32), jnp.full((_NPAD - _N,), _B, jnp.int32)]
    ).reshape(_NPAD // _RB, 1, _RB)

    degp = deg_fn(dst_p)
    deg0 = degp[0, :, 0:1]
    deg1 = degp[1, :, 0:1]
    dinv_col, xta, xtb = _stage_a(deg0, deg1, x_p)
    dinv_flat = dinv_col.reshape(_NPAD)

    s1a, s1b = agg64(xta, xtb, dinv_flat, src_p, dst_p)
    xt2a, xt2b = _stage_b(s1a, s1b, dinv_col, W1, b1.reshape(1, _H))
    s2a, s2b = agg128(xt2a, xt2b, dinv_flat, src_p, dst_p)

    wc_p = jnp.pad(Wc, ((0, 0), (0, 7)))
    outp = _stage_c(s2a, s2b, W2, b2.reshape(1, _H), wc_p,
                    bc.reshape(1, 1), batch_p)
    return outp[:, 0]


# speedup vs baseline: 5.7020x; 5.7020x over previous
import jax
import jax.numpy as jnp
from jax import lax
from jax.experimental import pallas as pl
from jax.experimental.pallas import tpu as pltpu
from jax.experimental.pallas import tpu_sc as plsc

_N = 10000
_E = 320000
_FIN = 128
_H = 256
_B = 64

_NC, _NS, _L = 2, 16, 16
_NPAD = 10240
_RPT = _NPAD // _NS
_CH = 128
_EPAD = 323584
_EPW = _EPAD // (_NC * _NS)
_EPS = _EPAD // _NS
_RB = 512


def _sc_mesh():
    return plsc.VectorSubcoreMesh(
        core_axis_name="c", subcore_axis_name="s",
        num_cores=_NC, num_subcores=_NS)


_SC_PARAMS = pltpu.CompilerParams(needs_layout_passes=False)


def _deg_body(dst_hbm, out_hbm, idxd, deg_v, tmp_v, acc_v, sbuf):
    c = lax.axis_index("c")
    s = lax.axis_index("s")
    w = s * _NC + c
    zeros = jnp.zeros((_L,), jnp.float32)
    ones = jnp.ones((_L,), jnp.float32)

    def _zero(i, carry):
        deg_v[pl.ds(i * _L, _L)] = zeros
        return carry
    lax.fori_loop(0, _NPAD // _L, _zero, 0)

    def _chunk(k, carry):
        base = w * _EPW + k * _CH
        pltpu.sync_copy(dst_hbm.at[pl.ds(base, _CH)], idxd)
        for j in range(_CH // _L):
            iv = idxd[pl.ds(j * _L, _L)]
            plsc.addupdate_scatter(deg_v, [iv], ones)
        return carry
    lax.fori_loop(0, _EPW // _CH, _chunk, 0)

    pltpu.sync_copy(deg_v, sbuf.at[s])
    plsc.subcore_barrier()

    pltpu.sync_copy(sbuf.at[0, pl.ds(s * _RPT, _RPT)], acc_v)

    def _red(t, carry):
        pltpu.sync_copy(sbuf.at[t, pl.ds(s * _RPT, _RPT)], tmp_v)

        def _add(i, carry2):
            sl = pl.ds(i * _L, _L)
            acc_v[sl] = acc_v[sl] + tmp_v[sl]
            return carry2
        lax.fori_loop(0, _RPT // _L, _add, 0)
        return carry
    lax.fori_loop(1, _NS, _red, 0)

    pltpu.sync_copy(acc_v, out_hbm.at[c, pl.ds(s * _RPT, _RPT)])


def _make_deg():
    return pl.kernel(
        _deg_body,
        out_type=jax.ShapeDtypeStruct((_NC, _NPAD), jnp.float32),
        mesh=_sc_mesh(),
        compiler_params=_SC_PARAMS,
        scratch_types=[
            pltpu.VMEM((_CH,), jnp.int32),
            pltpu.VMEM((_NPAD,), jnp.float32),
            pltpu.VMEM((_RPT,), jnp.float32),
            pltpu.VMEM((_RPT,), jnp.float32),
            pltpu.VMEM_SHARED((_NS, _NPAD), jnp.float32),
        ],
    )


_NHALF = _NPAD // 2
_AROWS = _NHALF + _CH
_GROW = _NHALF
_SEED = _NHALF // _NS


def _clamp_phase(idxd, p):
    for j in range(_CH // _L):
        sl = pl.ds(j * _L, _L)
        iv = idxd[sl]
        if p == 0:
            ivc = jnp.where(iv < _NHALF, iv, _GROW)
        else:
            ivc = jnp.where(iv >= _NHALF, iv - _NHALF, _GROW)
        idxd[sl] = ivc


def _agg_phases(s, nchunks, ebase, seed_from_xt, xt_hbm, s_hbm,
                src_hbm, dst_hbm, idxs, idxd, rows, xv, acc, sem):
    Fh = rows.shape[1]
    if not seed_from_xt:
        zeros = jnp.zeros((_L,), jnp.float32)

        def _z(i, carry):
            for j in range(Fh // _L):
                xv[i, pl.ds(j * _L, _L)] = zeros
            return carry
        lax.fori_loop(0, _SEED, _z, 0)

    for p in range(2):
        if seed_from_xt:
            pltpu.sync_copy(
                xt_hbm.at[pl.ds(p * _NHALF + s * _SEED, _SEED)],
                xv.at[pl.ds(0, _SEED)])
        pltpu.sync_copy(xv.at[pl.ds(0, _SEED)],
                        acc.at[pl.ds(s * _SEED, _SEED)])
        plsc.subcore_barrier()

        def _chunk(k, carry):
            base = ebase + k * _CH
            pltpu.sync_copy(src_hbm.at[pl.ds(base, _CH)], idxs)
            pltpu.async_copy(xt_hbm.at[idxs], rows, sem).wait()
            pltpu.sync_copy(dst_hbm.at[pl.ds(base, _CH)], idxd)
            _clamp_phase(idxd, p)
            pltpu.sync_copy(rows, acc.at[idxd], add=True)
            return carry
        lax.fori_loop(0, nchunks, _chunk, 0)

        plsc.subcore_barrier()
        pltpu.sync_copy(
            acc.at[pl.ds(s * _SEED, _SEED)],
            s_hbm.at[pl.ds(p * _NHALF + s * _SEED, _SEED)])


def _make_agg_edge(Fh):
    def _body(xt_hbm, src_hbm, dst_hbm, sa, sb, idxs, idxd, rows, xv, acc,
              sem):
        c = lax.axis_index("c")
        s = lax.axis_index("s")
        w = c * _NS + s

        @pl.when(c == 0)
        def _():
            _agg_phases(s, _EPW // _CH, w * _EPW, True, xt_hbm, sa,
                        src_hbm, dst_hbm, idxs, idxd, rows, xv, acc, sem)

        @pl.when(c == 1)
        def _():
            _agg_phases(s, _EPW // _CH, w * _EPW, False, xt_hbm, sb,
                        src_hbm, dst_hbm, idxs, idxd, rows, xv, acc, sem)

    return pl.kernel(
        _body,
        out_type=(jax.ShapeDtypeStruct((_NPAD, Fh), jnp.float32),
                  jax.ShapeDtypeStruct((_NPAD, Fh), jnp.float32)),
        mesh=_sc_mesh(),
        compiler_params=_SC_PARAMS,
        scratch_types=[
            pltpu.VMEM((_CH,), jnp.int32),
            pltpu.VMEM((_CH,), jnp.int32),
            pltpu.VMEM((_CH, Fh), jnp.float32),
            pltpu.VMEM((_SEED, Fh), jnp.float32),
            pltpu.VMEM_SHARED((_AROWS, Fh), jnp.float32),
            pltpu.SemaphoreType.DMA,
        ],
    )


def _make_agg(Fh):
    def _body(xta, xtb, src_hbm, dst_hbm, sa, sb,
              idxs, idxd, rows, xv, acc, sem):
        c = lax.axis_index("c")
        s = lax.axis_index("s")

        @pl.when(c == 0)
        def _():
            _agg_phases(s, _EPS // _CH, s * _EPS, True, xta, sa,
                        src_hbm, dst_hbm, idxs, idxd, rows, xv, acc, sem)

        @pl.when(c == 1)
        def _():
            _agg_phases(s, _EPS // _CH, s * _EPS, True, xtb, sb,
                        src_hbm, dst_hbm, idxs, idxd, rows, xv, acc, sem)

    return pl.kernel(
        _body,
        out_type=(jax.ShapeDtypeStruct((_NPAD, Fh), jnp.float32),
                  jax.ShapeDtypeStruct((_NPAD, Fh), jnp.float32)),
        mesh=_sc_mesh(),
        compiler_params=_SC_PARAMS,
        scratch_types=[
            pltpu.VMEM((_CH,), jnp.int32),
            pltpu.VMEM((_CH,), jnp.int32),
            pltpu.VMEM((_CH, Fh), jnp.float32),
            pltpu.VMEM((_SEED, Fh), jnp.float32),
            pltpu.VMEM_SHARED((_AROWS, Fh), jnp.float32),
            pltpu.SemaphoreType.DMA,
        ],
    )


def _stage_a_body(d0_ref, d1_ref, x_ref, dinv_ref, xt_ref):
    deg = d0_ref[...] + d1_ref[...] + 1.0
    dinv = lax.rsqrt(deg)
    dinv_ref[...] = dinv
    xt_ref[...] = x_ref[...] * dinv


_stage_a = pl.pallas_call(
    _stage_a_body,
    grid=(_NPAD // _RB,),
    in_specs=[
        pl.BlockSpec((_RB, 1), lambda i: (i, 0)),
        pl.BlockSpec((_RB, 1), lambda i: (i, 0)),
        pl.BlockSpec((_RB, _FIN), lambda i: (i, 0)),
    ],
    out_specs=[
        pl.BlockSpec((_RB, 1), lambda i: (i, 0)),
        pl.BlockSpec((_RB, _FIN), lambda i: (i, 0)),
    ],
    out_shape=[
        jax.ShapeDtypeStruct((_NPAD, 1), jnp.float32),
        jax.ShapeDtypeStruct((_NPAD, _FIN), jnp.float32),
    ],
)


def _stage_b_body(sa_ref, sb_ref, dinv_ref, w1_ref, b1_ref, xa_ref, xb_ref):
    agg = (sa_ref[...] + sb_ref[...]) * dinv_ref[...]
    h = jnp.dot(agg, w1_ref[...], preferred_element_type=jnp.float32)
    h = jnp.maximum(h + b1_ref[...], 0.0)
    xt2 = h * dinv_ref[...]
    xa_ref[...] = xt2[:, : _H // 2]
    xb_ref[...] = xt2[:, _H // 2:]


_stage_b = pl.pallas_call(
    _stage_b_body,
    grid=(_NPAD // _RB,),
    in_specs=[
        pl.BlockSpec((_RB, _FIN), lambda i: (i, 0)),
        pl.BlockSpec((_RB, _FIN), lambda i: (i, 0)),
        pl.BlockSpec((_RB, 1), lambda i: (i, 0)),
        pl.BlockSpec((_FIN, _H), lambda i: (0, 0)),
        pl.BlockSpec((1, _H), lambda i: (0, 0)),
    ],
    out_specs=[
        pl.BlockSpec((_RB, _H // 2), lambda i: (i, 0)),
        pl.BlockSpec((_RB, _H // 2), lambda i: (i, 0)),
    ],
    out_shape=[
        jax.ShapeDtypeStruct((_NPAD, _H // 2), jnp.float32),
        jax.ShapeDtypeStruct((_NPAD, _H // 2), jnp.float32),
    ],
)


def _stage_c_body(sa_ref, sb_ref, dinv_ref, w2_ref, b2_ref, wc_ref, bc_ref,
                  batch_ref, out_ref, cnt_ref):
    i = pl.program_id(0)
    agg = (jnp.concatenate([sa_ref[...], sb_ref[...]], axis=1)
           * dinv_ref[...])
    h2 = jnp.dot(agg, w2_ref[...], preferred_element_type=jnp.float32)
    h2 = jnp.maximum(h2 + b2_ref[...], 0.0)
    y = jnp.dot(h2, wc_ref[...], preferred_element_type=jnp.float32)
    b = batch_ref[0]
    rows = lax.broadcasted_iota(jnp.int32, (_B, _RB), 0)
    oh = (rows == b).astype(jnp.float32)
    part = jnp.dot(oh, y, preferred_element_type=jnp.float32)
    cpart = jnp.sum(oh, axis=1, keepdims=True)

    @pl.when(i == 0)
    def _():
        out_ref[...] = jnp.zeros_like(out_ref)
        cnt_ref[...] = jnp.zeros_like(cnt_ref)

    out_ref[...] += part
    cnt_ref[...] += cpart

    @pl.when(i == pl.num_programs(0) - 1)
    def _():
        out_ref[...] = (out_ref[...] / jnp.maximum(cnt_ref[...], 1.0)
                        + bc_ref[0, 0])


_stage_c = pl.pallas_call(
    _stage_c_body,
    grid=(_NPAD // _RB,),
    in_specs=[
        pl.BlockSpec((_RB, _H // 2), lambda i: (i, 0)),
        pl.BlockSpec((_RB, _H // 2), lambda i: (i, 0)),
        pl.BlockSpec((_RB, 1), lambda i: (i, 0)),
        pl.BlockSpec((_H, _H), lambda i: (0, 0)),
        pl.BlockSpec((1, _H), lambda i: (0, 0)),
        pl.BlockSpec((_H, 8), lambda i: (0, 0)),
        pl.BlockSpec((1, 1), lambda i: (0, 0)),
        pl.BlockSpec((1, 1, _RB), lambda i: (i, 0, 0)),
    ],
    out_specs=pl.BlockSpec((_B, 8), lambda i: (0, 0)),
    out_shape=jax.ShapeDtypeStruct((_B, 8), jnp.float32),
    scratch_shapes=[pltpu.VMEM((_B, 1), jnp.float32)],
)


_sc_kernels = None


def _get_sc_kernels():
    global _sc_kernels
    if _sc_kernels is None:
        _sc_kernels = (_make_deg(), _make_agg_edge(_FIN), _make_agg(_H // 2))
    return _sc_kernels


def kernel(x, edge_index, batch, W1, b1, W2, b2, Wc, bc):
    deg_fn, agg1, agg128 = _get_sc_kernels()
    src = edge_index[0].astype(jnp.int32)
    dst = edge_index[1].astype(jnp.int32)
    pad_e = _EPAD - _E
    src_p = jnp.concatenate([src, jnp.zeros((pad_e,), jnp.int32)])
    dst_p = jnp.concatenate([dst, jnp.full((pad_e,), _N, jnp.int32)])
    x_p = jnp.pad(x, ((0, _NPAD - _N), (0, 0)))
    batch_p = jnp.concatenate(
        [batch.astype(jnp.int32), jnp.full((_NPAD - _N,), _B, jnp.int32)]
    ).reshape(_NPAD // _RB, 1, _RB)

    degp = deg_fn(dst_p)
    deg0 = degp[0].reshape(_NPAD, 1)
    deg1 = degp[1].reshape(_NPAD, 1)
    dinv_col, xt1 = _stage_a(deg0, deg1, x_p)

    s1a, s1b = agg1(xt1, src_p, dst_p)
    xt2a, xt2b = _stage_b(s1a, s1b, dinv_col, W1, b1.reshape(1, _H))
    s2a, s2b = agg128(xt2a, xt2b, src_p, dst_p)

    wc_p = jnp.pad(Wc, ((0, 0), (0, 7)))
    outp = _stage_c(s2a, s2b, dinv_col, W2, b2.reshape(1, _H), wc_p,
                    bc.reshape(1, 1), batch_p)
    return outp[:, 0]
